# baseline (speedup 1.0000x reference)
import jax
import jax.numpy as jnp
from jax import lax
from jax.experimental import pallas as pl
from jax.experimental.pallas import tpu as pltpu
from jax.experimental.pallas import tpu_sc as plsc

VOCAB = 1000000
EMBED = 64
BATCH = 16384
CTX = 20
NPAIR = BATCH * CTX

NC = 2
NS = 16
NW = NC * NS
IPW = NPAIR // NW
BPW = BATCH // NW

BINW = 512
NBIN = 2048
BPWK = NBIN // NW
NE = EMBED // 16
TVC = 128
TVR = BATCH + 1
CAP = 345088
ZCAP = NPAIR + 16
EDGE = VOCAB - (VOCAB // BINW) * BINW
LASTBIN = VOCAB // BINW

_i32 = jnp.int32
_f32 = jnp.float32


def _lane(v, i):
    return jnp.squeeze(lax.slice(v, (i,), (i + 1,)))


def _wid():
    return lax.axis_index("s") * NC + lax.axis_index("c")


def _exscan(counts_hbm, cbuf, wid, mybase, totv, want_partial):
    widv = jnp.full((16,), wid, _i32)
    CHB = 256

    def chunk(jc, carry):
        cb = pl.multiple_of(jc * CHB, 8)
        for wp in range(NW):
            pltpu.sync_copy(counts_hbm.at[pl.ds(cb + wp * NBIN, CHB)],
                            cbuf.at[pl.ds(wp * CHB, CHB)])

        def j_body(j, carry2):
            base = pl.multiple_of(j * 16, 8)
            tot = jnp.zeros((16,), _i32)
            par = jnp.zeros((16,), _i32)
            for wp in range(NW):
                x = cbuf[pl.ds(wp * CHB + base, 16)]
                tot = tot + x
                if want_partial:
                    m = jnp.full((16,), wp, _i32) < widv
                    par = par + jnp.where(m, x, jnp.zeros((16,), _i32))
            tot8 = (tot + 7) & ~7
            incl = plsc.cumsum(tot8)
            excl = incl - tot8 + carry2
            totv[pl.ds(cb + base, 16)] = tot
            mybase[pl.ds(cb + base, 16)] = excl + par
            return carry2 + _lane(incl, 15)

        return lax.fori_loop(0, CHB // 16, j_body, carry)

    lax.fori_loop(0, NBIN // CHB, chunk, jnp.zeros((), _i32))


def _hist_body(ctx_hbm, counts_hbm, idxv, counters, sem):
    wid = _wid()
    pltpu.sync_copy(ctx_hbm.at[pl.ds(pl.multiple_of(wid * IPW, 8), IPW)], idxv)

    def zero(j, c):
        counters[pl.ds(pl.multiple_of(j * 16, 8), 16)] = jnp.zeros((16,), _i32)
        return c
    lax.fori_loop(0, NBIN // 16, zero, 0)

    def vec(v, c):
        x = idxv[pl.ds(pl.multiple_of(v * 16, 8), 16)]
        b = lax.shift_right_logical(x, 9)
        cnt, last = plsc.scan_count(b)
        plsc.addupdate_scatter(counters, [b], cnt, mask=last)
        return c
    lax.fori_loop(0, IPW // 16, vec, 0)
    pltpu.sync_copy(counters, counts_hbm.at[pl.ds(pl.multiple_of(wid * NBIN, 8), NBIN)])


def _scat_body(ctx_hbm, counts_hbm, bidx_hbm, bpid_hbm,
               idxv, cnts, mybase, totv, dstg, istg, pstg, sem):
    wid = _wid()
    lanes = lax.iota(_i32, 16)
    pltpu.sync_copy(ctx_hbm.at[pl.ds(pl.multiple_of(wid * IPW, 8), IPW)], idxv)
    _exscan(counts_hbm, cnts, wid, mybase, totv, True)

    def cg(g, c):
        for s in range(8):
            off = pl.multiple_of(g * 128 + s * 16, 8)
            x = idxv[pl.ds(off, 16)]
            pid = wid * IPW + off + lanes
            b = lax.shift_right_logical(x, 9)
            cnt, last = plsc.scan_count(b)
            base16 = plsc.load_gather(mybase, [b])
            dstg[pl.ds(s * 16, 16)] = base16 + cnt - 1
            istg[pl.ds(s * 16, 16)] = x
            pstg[pl.ds(s * 16, 16)] = pid
            plsc.addupdate_scatter(mybase, [b], cnt, mask=last)
        c1 = pltpu.async_copy(istg, bidx_hbm.at[dstg], sem)
        c2 = pltpu.async_copy(pstg, bpid_hbm.at[dstg], sem)
        c1.wait()
        c2.wait()
        return c
    lax.fori_loop(0, IPW // 128, cg, 0)


def _tgt_body(tgt_hbm, tt_hbm, ttedge_hbm, tv_hbm,
              tchunk, lbase, sfi, sfp, block, rowstg, pidstg, sem):
    wid = _wid()
    lanes = lax.iota(_i32, 16)
    widv = jnp.full((16,), wid, _i32)
    TC = 2048
    nv = TC // 16

    def zero(j, c):
        lbase[pl.ds(pl.multiple_of(j * 16, 8), 16)] = jnp.zeros((16,), _i32)
        return c
    lax.fori_loop(0, BPWK // 16, zero, 0)

    def p1(cidx, c):
        pltpu.sync_copy(tgt_hbm.at[pl.ds(pl.multiple_of(cidx * TC, 8), TC)], tchunk)

        def vec(v, c2):
            x = tchunk[pl.ds(v * 16, 16)]
            b = lax.shift_right_logical(x, 9)
            own = (b & 31) == widv
            bb = lax.shift_right_logical(x, 14)
            cnt, last = plsc.scan_count(bb, mask=own)
            plsc.addupdate_scatter(lbase, [bb], cnt, mask=last)
            return c2
        return lax.fori_loop(0, nv, vec, c)
    lax.fori_loop(0, BATCH // TC, p1, 0)

    def scan4(j, carry):
        jj = pl.multiple_of(j * 16, 8)
        x = lbase[pl.ds(jj, 16)]
        incl = plsc.cumsum(x)
        lbase[pl.ds(jj, 16)] = incl - x + carry
        return carry + _lane(incl, 15)
    lax.fori_loop(0, BPWK // 16, scan4, jnp.zeros((), _i32))

    def p2(cidx, c):
        pltpu.sync_copy(tgt_hbm.at[pl.ds(pl.multiple_of(cidx * TC, 8), TC)], tchunk)

        def vec(v, c2):
            x = tchunk[pl.ds(pl.multiple_of(v * 16, 8), 16)]
            pid = cidx * TC + v * 16 + lanes
            b = lax.shift_right_logical(x, 9)
            own = (b & 31) == widv
            bb = lax.shift_right_logical(x, 14)
            cnt, last = plsc.scan_count(bb, mask=own)
            base16 = plsc.load_gather(lbase, [bb])
            dst = base16 + cnt - 1
            plsc.store_scatter(sfi, [dst], x, mask=own)
            plsc.store_scatter(sfp, [dst], pid, mask=own)
            plsc.addupdate_scatter(lbase, [bb], cnt, mask=last)
            return c2
        return lax.fori_loop(0, nv, vec, c)
    lax.fori_loop(0, BATCH // TC, p2, 0)

    def resetpid(c):
        def rp(j, c2):
            pidstg[pl.ds(pl.multiple_of(j * 16, 8), 16)] = jnp.full((16,), BATCH, _i32)
            return c2
        return lax.fori_loop(0, 16, rp, c)
    resetpid(0)

    def bbloop(bb, stgrow):
        binid = bb * NW + wid
        col0 = pl.multiple_of(binid * BINW, 128)
        start = jnp.where(bb > 0,
                          _lane(plsc.load_gather(lbase, [jnp.full((16,), bb - 1, _i32)]), 0),
                          jnp.zeros((), _i32))
        cnt = _lane(plsc.load_gather(lbase, [jnp.full((16,), bb, _i32)]), 0) - start

        @pl.when(jnp.logical_and(binid < LASTBIN, cnt > 0))
        def _():
            pltpu.sync_copy(tt_hbm.at[:, pl.ds(col0, BINW)], block)

        @pl.when(jnp.logical_and(binid == LASTBIN, cnt > 0))
        def _():
            pltpu.sync_copy(ttedge_hbm, block.at[:, pl.ds(0, 128)])

        def item_vec(k, row):
            p0 = start + k * 16
            x = plsc.load_gather(sfi, [p0 + lanes])
            pid = plsc.load_gather(sfp, [p0 + lanes])
            nleft = cnt - k * 16
            valid = lanes < nleft
            cols = jnp.clip(x - col0, 0, BINW - 1)
            rows = jnp.full((16,), row, _i32) + lanes

            def flush(r):
                cp = pltpu.async_copy(rowstg, tv_hbm.at[pidstg], sem)
                cp.wait()
                return resetpid(jnp.zeros((), _i32)) * 0

            row2 = lax.cond(row >= 240, flush, lambda r: r, row)
            rows = jnp.full((16,), row2, _i32) + lanes
            for e in range(EMBED):
                vals = plsc.load_gather(block, [jnp.full((16,), e, _i32), cols])
                plsc.store_scatter(rowstg, [rows, jnp.full((16,), e, _i32)],
                                   vals, mask=valid)
            plsc.store_scatter(pidstg, [rows], pid, mask=valid)
            return row2 + jnp.minimum(jnp.maximum(nleft, 0), 16)

        nvb = lax.div(cnt + 15, 16)
        return lax.fori_loop(0, nvb, item_vec, stgrow)

    stgrow = lax.fori_loop(0, BPWK, bbloop, jnp.zeros((), _i32))

    @pl.when(stgrow > 0)
    def _():
        pltpu.async_copy(rowstg, tv_hbm.at[pidstg], sem).wait()


def _dot_body(bidx_hbm, bpid_hbm, counts_hbm, ct_hbm, ctedge_hbm, tv_hbm, z_hbm,
              cnts, cb8, totv, block, ivec, pvec, tloc, accs,
              zstg, pstg, sem, sem2, sem3):
    wid = _wid()
    lanes = lax.iota(_i32, 16)
    _exscan(counts_hbm, cnts, wid, cb8, totv, False)

    def bbloop(bb, c):
        binid = bb * NW + wid
        col0 = pl.multiple_of(binid * BINW, 128)
        bfull = jnp.full((16,), binid, _i32)
        start = _lane(plsc.load_gather(cb8, [bfull]), 0)
        cnt = _lane(plsc.load_gather(totv, [bfull]), 0)

        @pl.when(jnp.logical_and(binid < LASTBIN, cnt > 0))
        def _():
            pltpu.sync_copy(ct_hbm.at[:, pl.ds(col0, BINW)], block)

        @pl.when(jnp.logical_and(binid == LASTBIN, cnt > 0))
        def _():
            pltpu.sync_copy(ctedge_hbm, block.at[:, pl.ds(0, 128)])

        def chunk(ch, c2):
            pos = pl.multiple_of(start + ch * 256, 8)
            c1 = pltpu.async_copy(bidx_hbm.at[pl.ds(pos, 256)], ivec, sem)
            c2_ = pltpu.async_copy(bpid_hbm.at[pl.ds(pos, 256)], pvec, sem)
            c1.wait()
            c2_.wait()
            tcps = []
            for g in range(16):
                pidg = pvec[pl.ds(g * 16, 16)]
                b16 = jnp.clip(lax.div(pidg, CTX), 0, BATCH)
                tcps.append(pltpu.async_copy(tv_hbm.at[b16], tloc.at[g], sem2))
            for cp in tcps:
                cp.wait()
            zcps = []
            for g in range(16):
                pidg = pvec[pl.ds(g * 16, 16)]
                xg = ivec[pl.ds(g * 16, 16)]
                valid = (ch * 256 + g * 16 + lanes) < jnp.full((16,), cnt, _i32)
                cols = jnp.clip(xg - col0, 0, BINW - 1)
                for i in range(16):
                    cb = jnp.full((16,), _lane(cols, i), _i32)
                    acc = plsc.load_gather(block, [lanes, cb]) * tloc[g, i, pl.ds(0, 16)]
                    for k in range(1, NE):
                        acc = acc + (plsc.load_gather(block, [k * 16 + lanes, cb])
                                     * tloc[g, i, pl.ds(k * 16, 16)])
                    accs[i, :] = acc
                zvec = plsc.load_gather(accs, [lanes, jnp.zeros((16,), _i32)])
                for l in range(1, 16):
                    zvec = zvec + plsc.load_gather(
                        accs, [lanes, jnp.full((16,), l, _i32)])
                s = g % 8
                zstg[pl.ds(s * 16, 16)] = zvec
                pstg[pl.ds(s * 16, 16)] = jnp.where(
                    valid, pidg, jnp.full((16,), NPAIR, _i32) + lanes)
                if g % 8 == 7:
                    zcps.append(pltpu.async_copy(zstg, z_hbm.at[pstg], sem3))
                    zcps[-1].wait()
            return c2
        nch = lax.div(cnt + 255, 256)
        return lax.fori_loop(0, nch, chunk, c)

    lax.fori_loop(0, BPWK, bbloop, 0)


def _mkparams():
    return pltpu.CompilerParams(
        needs_layout_passes=False, use_tc_tiling_on_sc=True)


def kernel(targets, contexts, target_table, context_table):
    mesh = plsc.VectorSubcoreMesh(core_axis_name="c", subcore_axis_name="s")
    ctxf = contexts.reshape(-1).astype(_i32)
    tgts = targets.astype(_i32)

    counts = pl.kernel(
        _hist_body,
        out_type=jax.ShapeDtypeStruct((NW * NBIN,), _i32),
        mesh=mesh, compiler_params=_mkparams(),
        scratch_types=[
            pltpu.VMEM((IPW,), _i32),
            pltpu.VMEM((NBIN,), _i32),
            pltpu.SemaphoreType.DMA,
        ],
    )(ctxf)

    bidx, bpid = pl.kernel(
        _scat_body,
        out_type=(jax.ShapeDtypeStruct((CAP,), _i32),
                  jax.ShapeDtypeStruct((CAP,), _i32)),
        mesh=mesh, compiler_params=_mkparams(),
        scratch_types=[
            pltpu.VMEM((IPW,), _i32),
            pltpu.VMEM((NW * 256,), _i32),
            pltpu.VMEM((NBIN,), _i32),
            pltpu.VMEM((NBIN,), _i32),
            pltpu.VMEM((128,), _i32),
            pltpu.VMEM((128,), _i32),
            pltpu.VMEM((128,), _i32),
            pltpu.SemaphoreType.DMA,
        ],
    )(ctxf, counts)

    tvecs = pl.kernel(
        _tgt_body,
        out_type=jax.ShapeDtypeStruct((TVR, TVC), _f32),
        mesh=mesh, compiler_params=_mkparams(),
        scratch_types=[
            pltpu.VMEM((2048,), _i32),
            pltpu.VMEM((BPWK,), _i32),
            pltpu.VMEM((BATCH,), _i32),
            pltpu.VMEM((BATCH,), _i32),
            pltpu.VMEM((EMBED, BINW), _f32),
            pltpu.VMEM((256, TVC), _f32),
            pltpu.VMEM((256,), _i32),
            pltpu.SemaphoreType.DMA,
        ],
    )(tgts, target_table.T,
      jnp.pad(target_table.T[:, LASTBIN * BINW:], ((0, 0), (0, 128 - EDGE))))

    z = pl.kernel(
        _dot_body,
        out_type=jax.ShapeDtypeStruct((ZCAP,), _f32),
        mesh=mesh, compiler_params=_mkparams(),
        scratch_types=[
            pltpu.VMEM((NW * 256,), _i32),
            pltpu.VMEM((NBIN,), _i32),
            pltpu.VMEM((NBIN,), _i32),
            pltpu.VMEM((EMBED, BINW), _f32),
            pltpu.VMEM((256,), _i32),
            pltpu.VMEM((256,), _i32),
            pltpu.VMEM((16, 16, TVC), _f32),
            pltpu.VMEM((16, 16), _f32),
            pltpu.VMEM((128,), _f32),
            pltpu.VMEM((128,), _i32),
            pltpu.SemaphoreType.DMA,
            pltpu.SemaphoreType.DMA,
            pltpu.SemaphoreType.DMA,
        ],
    )(bidx, bpid, counts, context_table.T,
      jnp.pad(context_table.T[:, LASTBIN * BINW:], ((0, 0), (0, 128 - EDGE))),
      tvecs)

    return z[:NPAIR].reshape(BATCH, CTX)
